# baseline (speedup 1.0000x reference)
import functools

import jax
import jax.numpy as jnp
from jax import lax
from jax.experimental import pallas as pl
from jax.experimental.pallas import tpu as pltpu
from jax.experimental.pallas import tpu_sc as plsc

L = 4
B = 8
H = 8
MAX_LEN = 4096
D = 128
T = 32
PREFIX = 2048 + T
BH = B * H
NW = 32
UNITS_PER_W = BH // NW
CHUNK = 416
NCHUNK = PREFIX // CHUNK
TC_NBUF = 8
TC_PREFETCH = 4


def _make_sc_copy():
    mesh = plsc.VectorSubcoreMesh(core_axis_name="c", subcore_axis_name="s")

    @functools.partial(
        pl.kernel,
        mesh=mesh,
        out_type=jax.ShapeDtypeStruct((BH * PREFIX, D), jnp.float32),
        scratch_types=[
            pltpu.VMEM((16,), jnp.int32),
            pltpu.VMEM((CHUNK, D), jnp.float32),
            pltpu.VMEM((CHUNK, D), jnp.float32),
            pltpu.SemaphoreType.DMA,
            pltpu.SemaphoreType.DMA,
            pltpu.SemaphoreType.DMA,
            pltpu.SemaphoreType.DMA,
        ],
    )
    def sc_copy(ks, kn, meta_hbm, k_out,
                meta_v, buf_a, buf_b, in_a, in_b, out_a, out_b):
        bufs = (buf_a, buf_b)
        sin = (in_a, in_b)
        sout = (out_a, out_b)
        wid = lax.axis_index("s") * 2 + lax.axis_index("c")
        pltpu.sync_copy(meta_hbm, meta_v)
        meta = meta_v[...]
        layer_base = pl.multiple_of(meta[0] * (B * H * MAX_LEN), 8)
        pos = pl.multiple_of(meta[1], 8)

        chunks = []
        for j in range(UNITS_PER_W):
            bh = wid * UNITS_PER_W + j
            src_base = layer_base + bh * MAX_LEN
            dst_base = bh * PREFIX
            for c in range(NCHUNK):
                chunks.append((src_base + c * CHUNK, dst_base + c * CHUNK,
                               bh * T, c * CHUNK))

        n = len(chunks)
        pend_store = [None, None]
        pend_load = [None, None]

        def start_load(i):
            b = i % 2
            if pend_store[b] is not None:
                pend_store[b].wait()
                pend_store[b] = None
            pend_load[b] = pltpu.async_copy(
                ks.at[pl.ds(pl.multiple_of(chunks[i][0], 8), CHUNK)],
                bufs[b], sin[b])

        start_load(0)
        for i in range(n):
            b = i % 2
            if i + 1 < n:
                start_load(i + 1)
            pend_load[b].wait()
            _, dst_row, new_row, rel = chunks[i]
            delta = pos - rel
            @pl.when(jnp.logical_and(delta >= 0, delta <= CHUNK - T))
            def _():
                pltpu.sync_copy(
                    kn.at[pl.ds(new_row, T)],
                    bufs[b].at[pl.ds(pl.multiple_of(delta, 8), T)])
            pend_store[b] = pltpu.async_copy(
                bufs[b], k_out.at[pl.ds(pl.multiple_of(dst_row, 8), CHUNK)],
                sout[b])
        for b in range(2):
            if pend_store[b] is not None:
                pend_store[b].wait()

    return sc_copy


def _tc_copy_body(meta_ref, vs, vn, v_out, vn_buf, *bufs_sems):
    bufs = bufs_sems[:TC_NBUF]
    sin = bufs_sems[TC_NBUF]
    sout = bufs_sems[TC_NBUF + 1]
    svn = bufs_sems[TC_NBUF + 2]
    layer_base = pl.multiple_of(meta_ref[0] * (B * H * MAX_LEN), 8)
    pos = pl.multiple_of(meta_ref[1], 8)

    pend_store = [None] * TC_NBUF
    pend_load = [None] * TC_NBUF

    vn_c = pltpu.make_async_copy(vn, vn_buf, svn)
    vn_c.start()

    def start_load(u):
        b = u % TC_NBUF
        if pend_store[b] is not None:
            pend_store[b].wait()
            pend_store[b] = None
        c = pltpu.make_async_copy(
            vs.at[pl.ds(pl.multiple_of(layer_base + u * MAX_LEN, 8), PREFIX)],
            bufs[b], sin.at[b])
        c.start()
        pend_load[b] = c

    for u in range(min(TC_PREFETCH, BH)):
        start_load(u)
    vn_c.wait()
    for u in range(BH):
        b = u % TC_NBUF
        if u + TC_PREFETCH < BH:
            start_load(u + TC_PREFETCH)
        pend_load[b].wait()
        bufs[b][pl.ds(pos, T), :] = vn_buf[pl.ds(u * T, T), :]
        st = pltpu.make_async_copy(
            bufs[b], v_out.at[pl.ds(u * PREFIX, PREFIX)], sout.at[b])
        st.start()
        pend_store[b] = st
    for b in range(TC_NBUF):
        if pend_store[b] is not None:
            pend_store[b].wait()


def _make_tc_copy():
    return pl.pallas_call(
        _tc_copy_body,
        in_specs=[
            pl.BlockSpec(memory_space=pltpu.SMEM),
            pl.BlockSpec(memory_space=pl.ANY),
            pl.BlockSpec(memory_space=pl.ANY),
        ],
        out_specs=pl.BlockSpec(memory_space=pl.ANY),
        out_shape=jax.ShapeDtypeStruct((BH * PREFIX, D), jnp.float32),
        scratch_shapes=(
            [pltpu.VMEM((BH * T, D), jnp.float32)]
            + [pltpu.VMEM((PREFIX, D), jnp.float32) for _ in range(TC_NBUF)]
            + [pltpu.SemaphoreType.DMA((TC_NBUF,)),
               pltpu.SemaphoreType.DMA((TC_NBUF,)),
               pltpu.SemaphoreType.DMA]
        ),
    )


_SC_COPY = _make_sc_copy()
_TC_COPY = _make_tc_copy()


def kernel(k_state, v_state, k_new, v_new, layer_idx, pos):
    ks = k_state.reshape(L * BH * MAX_LEN, D)
    vs = v_state.reshape(L * BH * MAX_LEN, D)
    kn = k_new.reshape(BH * T, D)
    vn = v_new.reshape(BH * T, D)
    meta = jnp.zeros((16,), jnp.int32)
    meta = meta.at[0].set(jnp.asarray(layer_idx, jnp.int32))
    meta = meta.at[1].set(jnp.asarray(pos, jnp.int32))
    k_out = _SC_COPY(ks, kn, meta)
    v_out = _TC_COPY(meta, vs, vn)
    return (
        k_out.reshape(B, H, PREFIX, D),
        v_out.reshape(B, H, PREFIX, D),
    )


# speedup vs baseline: 1.0049x; 1.0049x over previous
import functools

import jax
import jax.numpy as jnp
from jax import lax
from jax.experimental import pallas as pl
from jax.experimental.pallas import tpu as pltpu
from jax.experimental.pallas import tpu_sc as plsc

L = 4
B = 8
H = 8
MAX_LEN = 4096
D = 128
T = 32
PREFIX = 2048 + T
BH = B * H
NW = 32
UNITS_PER_W = BH // NW
CHUNK = 416
NCHUNK = PREFIX // CHUNK
TC_NBUF = 6
TC_PREFETCH = 3


def _make_sc_copy():
    mesh = plsc.VectorSubcoreMesh(core_axis_name="c", subcore_axis_name="s")

    @functools.partial(
        pl.kernel,
        mesh=mesh,
        out_type=jax.ShapeDtypeStruct((BH * PREFIX, D), jnp.float32),
        scratch_types=[
            pltpu.VMEM((16,), jnp.int32),
            pltpu.VMEM((CHUNK, D), jnp.float32),
            pltpu.VMEM((CHUNK, D), jnp.float32),
            pltpu.SemaphoreType.DMA,
            pltpu.SemaphoreType.DMA,
            pltpu.SemaphoreType.DMA,
            pltpu.SemaphoreType.DMA,
        ],
    )
    def sc_copy(ks, kn, meta_hbm, k_out,
                meta_v, buf_a, buf_b, in_a, in_b, out_a, out_b):
        bufs = (buf_a, buf_b)
        sin = (in_a, in_b)
        sout = (out_a, out_b)
        wid = lax.axis_index("s") * 2 + lax.axis_index("c")
        pltpu.sync_copy(meta_hbm, meta_v)
        meta = meta_v[...]
        layer_base = pl.multiple_of(meta[0] * (B * H * MAX_LEN), 8)
        pos = pl.multiple_of(meta[1], 8)

        chunks = []
        for j in range(UNITS_PER_W):
            bh = wid * UNITS_PER_W + j
            src_base = layer_base + bh * MAX_LEN
            dst_base = bh * PREFIX
            for c in range(NCHUNK):
                chunks.append((src_base + c * CHUNK, dst_base + c * CHUNK,
                               bh * T, c * CHUNK))

        n = len(chunks)
        pend_store = [None, None]
        pend_load = [None, None]

        def start_load(i):
            b = i % 2
            if pend_store[b] is not None:
                pend_store[b].wait()
                pend_store[b] = None
            pend_load[b] = pltpu.async_copy(
                ks.at[pl.ds(pl.multiple_of(chunks[i][0], 8), CHUNK)],
                bufs[b], sin[b])

        start_load(0)
        for i in range(n):
            b = i % 2
            if i + 1 < n:
                start_load(i + 1)
            pend_load[b].wait()
            _, dst_row, new_row, rel = chunks[i]
            delta = pos - rel
            @pl.when(jnp.logical_and(delta >= 0, delta <= CHUNK - T))
            def _():
                pltpu.sync_copy(
                    kn.at[pl.ds(new_row, T)],
                    bufs[b].at[pl.ds(pl.multiple_of(delta, 8), T)])
            pend_store[b] = pltpu.async_copy(
                bufs[b], k_out.at[pl.ds(pl.multiple_of(dst_row, 8), CHUNK)],
                sout[b])
        for b in range(2):
            if pend_store[b] is not None:
                pend_store[b].wait()

    return sc_copy


def _tc_copy_body(meta_ref, vs, vn, v_out, vn_buf, *bufs_sems):
    bufs = bufs_sems[:TC_NBUF]
    sin = bufs_sems[TC_NBUF]
    sout = bufs_sems[TC_NBUF + 1]
    svn = bufs_sems[TC_NBUF + 2]
    layer_base = pl.multiple_of(meta_ref[0] * (B * H * MAX_LEN), 8)
    pos = pl.multiple_of(meta_ref[1], 8)

    pend_store = [None] * TC_NBUF
    pend_load = [None] * TC_NBUF

    vn_c = pltpu.make_async_copy(vn, vn_buf, svn)
    vn_c.start()

    def start_load(u):
        b = u % TC_NBUF
        if pend_store[b] is not None:
            pend_store[b].wait()
            pend_store[b] = None
        c = pltpu.make_async_copy(
            vs.at[pl.ds(pl.multiple_of(layer_base + u * MAX_LEN, 8), PREFIX)],
            bufs[b], sin.at[b])
        c.start()
        pend_load[b] = c

    for u in range(min(TC_PREFETCH, BH)):
        start_load(u)
    vn_c.wait()
    for u in range(BH):
        b = u % TC_NBUF
        if u + TC_PREFETCH < BH:
            start_load(u + TC_PREFETCH)
        pend_load[b].wait()
        bufs[b][pl.ds(pos, T), :] = vn_buf[pl.ds(u * T, T), :]
        st = pltpu.make_async_copy(
            bufs[b], v_out.at[pl.ds(u * PREFIX, PREFIX)], sout.at[b])
        st.start()
        pend_store[b] = st
    for b in range(TC_NBUF):
        if pend_store[b] is not None:
            pend_store[b].wait()


def _make_tc_copy():
    return pl.pallas_call(
        _tc_copy_body,
        in_specs=[
            pl.BlockSpec(memory_space=pltpu.SMEM),
            pl.BlockSpec(memory_space=pl.ANY),
            pl.BlockSpec(memory_space=pl.ANY),
        ],
        out_specs=pl.BlockSpec(memory_space=pl.ANY),
        out_shape=jax.ShapeDtypeStruct((BH * PREFIX, D), jnp.float32),
        scratch_shapes=(
            [pltpu.VMEM((BH * T, D), jnp.float32)]
            + [pltpu.VMEM((PREFIX, D), jnp.float32) for _ in range(TC_NBUF)]
            + [pltpu.SemaphoreType.DMA((TC_NBUF,)),
               pltpu.SemaphoreType.DMA((TC_NBUF,)),
               pltpu.SemaphoreType.DMA]
        ),
    )


_SC_COPY = _make_sc_copy()
_TC_COPY = _make_tc_copy()


def kernel(k_state, v_state, k_new, v_new, layer_idx, pos):
    ks = k_state.reshape(L * BH * MAX_LEN, D)
    vs = v_state.reshape(L * BH * MAX_LEN, D)
    kn = k_new.reshape(BH * T, D)
    vn = v_new.reshape(BH * T, D)
    meta = jnp.zeros((16,), jnp.int32)
    meta = meta.at[0].set(jnp.asarray(layer_idx, jnp.int32))
    meta = meta.at[1].set(jnp.asarray(pos, jnp.int32))
    k_out = _SC_COPY(ks, kn, meta)
    v_out = _TC_COPY(meta, vs, vn)
    return (
        k_out.reshape(B, H, PREFIX, D),
        v_out.reshape(B, H, PREFIX, D),
    )


# speedup vs baseline: 1.0100x; 1.0050x over previous
import functools

import jax
import jax.numpy as jnp
from jax import lax
from jax.experimental import pallas as pl
from jax.experimental.pallas import tpu as pltpu
from jax.experimental.pallas import tpu_sc as plsc

L = 4
B = 8
H = 8
MAX_LEN = 4096
D = 128
T = 32
PREFIX = 2048 + T
BH = B * H
NW = 32
UNITS_PER_W = BH // NW
CHUNK = 208
NCHUNK = PREFIX // CHUNK
SC_NBUF = 4
SC_PREFETCH = 2
TC_NBUF = 6
TC_PREFETCH = 3


def _make_sc_copy():
    mesh = plsc.VectorSubcoreMesh(core_axis_name="c", subcore_axis_name="s")

    @functools.partial(
        pl.kernel,
        mesh=mesh,
        out_type=jax.ShapeDtypeStruct((BH * PREFIX, D), jnp.float32),
        scratch_types=[
            pltpu.VMEM((16,), jnp.int32),
        ] + [pltpu.VMEM((CHUNK, D), jnp.float32) for _ in range(SC_NBUF)]
          + [pltpu.SemaphoreType.DMA for _ in range(2 * SC_NBUF)],
    )
    def sc_copy(ks, kn, meta_hbm, k_out, meta_v, *bufs_sems):
        bufs = bufs_sems[:SC_NBUF]
        sin = bufs_sems[SC_NBUF:2 * SC_NBUF]
        sout = bufs_sems[2 * SC_NBUF:]
        wid = lax.axis_index("s") * 2 + lax.axis_index("c")
        pltpu.sync_copy(meta_hbm, meta_v)
        meta = meta_v[...]
        layer_base = pl.multiple_of(meta[0] * (B * H * MAX_LEN), 8)
        pos = pl.multiple_of(meta[1], 8)

        chunks = []
        for j in range(UNITS_PER_W):
            bh = wid * UNITS_PER_W + j
            src_base = layer_base + bh * MAX_LEN
            dst_base = bh * PREFIX
            for c in range(NCHUNK):
                chunks.append((src_base + c * CHUNK, dst_base + c * CHUNK,
                               bh * T, c * CHUNK))

        n = len(chunks)
        pend_store = [None] * SC_NBUF
        pend_load = [None] * SC_NBUF

        def start_load(i):
            b = i % SC_NBUF
            if pend_store[b] is not None:
                pend_store[b].wait()
                pend_store[b] = None
            pend_load[b] = pltpu.async_copy(
                ks.at[pl.ds(pl.multiple_of(chunks[i][0], 8), CHUNK)],
                bufs[b], sin[b])

        for i in range(min(SC_PREFETCH, n)):
            start_load(i)
        for i in range(n):
            b = i % SC_NBUF
            if i + SC_PREFETCH < n:
                start_load(i + SC_PREFETCH)
            pend_load[b].wait()
            _, dst_row, new_row, rel = chunks[i]
            delta = pos - rel
            @pl.when(jnp.logical_and(delta >= 0, delta <= CHUNK - T))
            def _():
                pltpu.sync_copy(
                    kn.at[pl.ds(new_row, T)],
                    bufs[b].at[pl.ds(pl.multiple_of(delta, 8), T)])
            pend_store[b] = pltpu.async_copy(
                bufs[b], k_out.at[pl.ds(pl.multiple_of(dst_row, 8), CHUNK)],
                sout[b])
        for b in range(SC_NBUF):
            if pend_store[b] is not None:
                pend_store[b].wait()

    return sc_copy


def _tc_copy_body(meta_ref, vs, vn, v_out, vn_buf, *bufs_sems):
    bufs = bufs_sems[:TC_NBUF]
    sin = bufs_sems[TC_NBUF]
    sout = bufs_sems[TC_NBUF + 1]
    svn = bufs_sems[TC_NBUF + 2]
    layer_base = pl.multiple_of(meta_ref[0] * (B * H * MAX_LEN), 8)
    pos = pl.multiple_of(meta_ref[1], 8)

    pend_store = [None] * TC_NBUF
    pend_load = [None] * TC_NBUF

    vn_c = pltpu.make_async_copy(vn, vn_buf, svn)
    vn_c.start()

    def start_load(u):
        b = u % TC_NBUF
        if pend_store[b] is not None:
            pend_store[b].wait()
            pend_store[b] = None
        c = pltpu.make_async_copy(
            vs.at[pl.ds(pl.multiple_of(layer_base + u * MAX_LEN, 8), PREFIX)],
            bufs[b], sin.at[b])
        c.start()
        pend_load[b] = c

    for u in range(min(TC_PREFETCH, BH)):
        start_load(u)
    vn_c.wait()
    for u in range(BH):
        b = u % TC_NBUF
        if u + TC_PREFETCH < BH:
            start_load(u + TC_PREFETCH)
        pend_load[b].wait()
        bufs[b][pl.ds(pos, T), :] = vn_buf[pl.ds(u * T, T), :]
        st = pltpu.make_async_copy(
            bufs[b], v_out.at[pl.ds(u * PREFIX, PREFIX)], sout.at[b])
        st.start()
        pend_store[b] = st
    for b in range(TC_NBUF):
        if pend_store[b] is not None:
            pend_store[b].wait()


def _make_tc_copy():
    return pl.pallas_call(
        _tc_copy_body,
        in_specs=[
            pl.BlockSpec(memory_space=pltpu.SMEM),
            pl.BlockSpec(memory_space=pl.ANY),
            pl.BlockSpec(memory_space=pl.ANY),
        ],
        out_specs=pl.BlockSpec(memory_space=pl.ANY),
        out_shape=jax.ShapeDtypeStruct((BH * PREFIX, D), jnp.float32),
        scratch_shapes=(
            [pltpu.VMEM((BH * T, D), jnp.float32)]
            + [pltpu.VMEM((PREFIX, D), jnp.float32) for _ in range(TC_NBUF)]
            + [pltpu.SemaphoreType.DMA((TC_NBUF,)),
               pltpu.SemaphoreType.DMA((TC_NBUF,)),
               pltpu.SemaphoreType.DMA]
        ),
    )


_SC_COPY = _make_sc_copy()
_TC_COPY = _make_tc_copy()


def kernel(k_state, v_state, k_new, v_new, layer_idx, pos):
    ks = k_state.reshape(L * BH * MAX_LEN, D)
    vs = v_state.reshape(L * BH * MAX_LEN, D)
    kn = k_new.reshape(BH * T, D)
    vn = v_new.reshape(BH * T, D)
    meta = jnp.zeros((16,), jnp.int32)
    meta = meta.at[0].set(jnp.asarray(layer_idx, jnp.int32))
    meta = meta.at[1].set(jnp.asarray(pos, jnp.int32))
    k_out = _SC_COPY(ks, kn, meta)
    v_out = _TC_COPY(meta, vs, vn)
    return (
        k_out.reshape(B, H, PREFIX, D),
        v_out.reshape(B, H, PREFIX, D),
    )


# speedup vs baseline: 1.0136x; 1.0036x over previous
import functools

import jax
import jax.numpy as jnp
from jax import lax
from jax.experimental import pallas as pl
from jax.experimental.pallas import tpu as pltpu
from jax.experimental.pallas import tpu_sc as plsc

L = 4
B = 8
H = 8
MAX_LEN = 4096
D = 128
T = 32
PREFIX = 2048 + T
BH = B * H
NW = 32
UNITS_PER_W = BH // NW
CHUNK = 208
NCHUNK = PREFIX // CHUNK
SC_NBUF = 4
SC_PREFETCH = 3
TC_NBUF = 6
TC_PREFETCH = 3


def _make_sc_copy():
    mesh = plsc.VectorSubcoreMesh(core_axis_name="c", subcore_axis_name="s")

    @functools.partial(
        pl.kernel,
        mesh=mesh,
        out_type=jax.ShapeDtypeStruct((BH * PREFIX, D), jnp.float32),
        scratch_types=[
            pltpu.VMEM((16,), jnp.int32),
        ] + [pltpu.VMEM((CHUNK, D), jnp.float32) for _ in range(SC_NBUF)]
          + [pltpu.SemaphoreType.DMA for _ in range(2 * SC_NBUF)],
    )
    def sc_copy(ks, kn, meta_hbm, k_out, meta_v, *bufs_sems):
        bufs = bufs_sems[:SC_NBUF]
        sin = bufs_sems[SC_NBUF:2 * SC_NBUF]
        sout = bufs_sems[2 * SC_NBUF:]
        wid = lax.axis_index("s") * 2 + lax.axis_index("c")
        pltpu.sync_copy(meta_hbm, meta_v)
        meta = meta_v[...]
        layer_base = pl.multiple_of(meta[0] * (B * H * MAX_LEN), 8)
        pos = pl.multiple_of(meta[1], 8)

        chunks = []
        for j in range(UNITS_PER_W):
            bh = wid * UNITS_PER_W + j
            src_base = layer_base + bh * MAX_LEN
            dst_base = bh * PREFIX
            for c in range(NCHUNK):
                chunks.append((src_base + c * CHUNK, dst_base + c * CHUNK,
                               bh * T, c * CHUNK))

        n = len(chunks)
        pend_store = [None] * SC_NBUF
        pend_load = [None] * SC_NBUF

        def start_load(i):
            b = i % SC_NBUF
            if pend_store[b] is not None:
                pend_store[b].wait()
                pend_store[b] = None
            pend_load[b] = pltpu.async_copy(
                ks.at[pl.ds(pl.multiple_of(chunks[i][0], 8), CHUNK)],
                bufs[b], sin[b])

        for i in range(min(SC_PREFETCH, n)):
            start_load(i)
        for i in range(n):
            b = i % SC_NBUF
            if i + SC_PREFETCH < n:
                start_load(i + SC_PREFETCH)
            pend_load[b].wait()
            _, dst_row, new_row, rel = chunks[i]
            delta = pos - rel
            @pl.when(jnp.logical_and(delta >= 0, delta <= CHUNK - T))
            def _():
                pltpu.sync_copy(
                    kn.at[pl.ds(new_row, T)],
                    bufs[b].at[pl.ds(pl.multiple_of(delta, 8), T)])
            pend_store[b] = pltpu.async_copy(
                bufs[b], k_out.at[pl.ds(pl.multiple_of(dst_row, 8), CHUNK)],
                sout[b])
        for b in range(SC_NBUF):
            if pend_store[b] is not None:
                pend_store[b].wait()

    return sc_copy


def _tc_copy_body(meta_ref, vs, vn, v_out, vn_buf, *bufs_sems):
    bufs = bufs_sems[:TC_NBUF]
    sin = bufs_sems[TC_NBUF]
    sout = bufs_sems[TC_NBUF + 1]
    svn = bufs_sems[TC_NBUF + 2]
    layer_base = pl.multiple_of(meta_ref[0] * (B * H * MAX_LEN), 8)
    pos = pl.multiple_of(meta_ref[1], 8)

    pend_store = [None] * TC_NBUF
    pend_load = [None] * TC_NBUF

    vn_c = pltpu.make_async_copy(vn, vn_buf, svn)
    vn_c.start()

    def start_load(u):
        b = u % TC_NBUF
        if pend_store[b] is not None:
            pend_store[b].wait()
            pend_store[b] = None
        c = pltpu.make_async_copy(
            vs.at[pl.ds(pl.multiple_of(layer_base + u * MAX_LEN, 8), PREFIX)],
            bufs[b], sin.at[b])
        c.start()
        pend_load[b] = c

    for u in range(min(TC_PREFETCH, BH)):
        start_load(u)
    vn_c.wait()
    for u in range(BH):
        b = u % TC_NBUF
        if u + TC_PREFETCH < BH:
            start_load(u + TC_PREFETCH)
        pend_load[b].wait()
        bufs[b][pl.ds(pos, T), :] = vn_buf[pl.ds(u * T, T), :]
        st = pltpu.make_async_copy(
            bufs[b], v_out.at[pl.ds(u * PREFIX, PREFIX)], sout.at[b])
        st.start()
        pend_store[b] = st
    for b in range(TC_NBUF):
        if pend_store[b] is not None:
            pend_store[b].wait()


def _make_tc_copy():
    return pl.pallas_call(
        _tc_copy_body,
        in_specs=[
            pl.BlockSpec(memory_space=pltpu.SMEM),
            pl.BlockSpec(memory_space=pl.ANY),
            pl.BlockSpec(memory_space=pl.ANY),
        ],
        out_specs=pl.BlockSpec(memory_space=pl.ANY),
        out_shape=jax.ShapeDtypeStruct((BH * PREFIX, D), jnp.float32),
        scratch_shapes=(
            [pltpu.VMEM((BH * T, D), jnp.float32)]
            + [pltpu.VMEM((PREFIX, D), jnp.float32) for _ in range(TC_NBUF)]
            + [pltpu.SemaphoreType.DMA((TC_NBUF,)),
               pltpu.SemaphoreType.DMA((TC_NBUF,)),
               pltpu.SemaphoreType.DMA]
        ),
    )


_SC_COPY = _make_sc_copy()
_TC_COPY = _make_tc_copy()


def kernel(k_state, v_state, k_new, v_new, layer_idx, pos):
    ks = k_state.reshape(L * BH * MAX_LEN, D)
    vs = v_state.reshape(L * BH * MAX_LEN, D)
    kn = k_new.reshape(BH * T, D)
    vn = v_new.reshape(BH * T, D)
    meta = jnp.zeros((16,), jnp.int32)
    meta = meta.at[0].set(jnp.asarray(layer_idx, jnp.int32))
    meta = meta.at[1].set(jnp.asarray(pos, jnp.int32))
    k_out = _SC_COPY(ks, kn, meta)
    v_out = _TC_COPY(meta, vs, vn)
    return (
        k_out.reshape(B, H, PREFIX, D),
        v_out.reshape(B, H, PREFIX, D),
    )
